# Initial kernel scaffold; baseline (speedup 1.0000x reference)
import jax
import jax.numpy as jnp
from jax.experimental import pallas as pl


def kernel(hidden_states, x_pack_kwargs, W_pre, W_res, w_router, W1, W2, W_post):
    raise NotImplementedError("write your pallas kernel here")


# speedup vs baseline: 1.7093x; 1.7093x over previous
import functools

import jax
import jax.numpy as jnp
from jax.experimental import pallas as pl
from jax.experimental.pallas import tpu as pltpu


def _fused_block(x_ref, wpre_ref, wres_ref, wrt_ref, w1_ref, w2_ref,
                 wpost_ref, out_ref, carry_ref, *, lb):
    i = pl.program_id(0)
    f32 = jnp.float32

    x = x_ref[...]
    h = jax.nn.gelu(jnp.dot(x, wpre_ref[...], preferred_element_type=f32))
    res = jnp.dot(h, wres_ref[...], preferred_element_type=f32)
    logits = jnp.dot(h, wrt_ref[...], preferred_element_type=f32)
    probs = jax.nn.sigmoid(logits)

    row = jax.lax.broadcasted_iota(jnp.int32, (lb, 1), 0)
    mask = (probs >= 0.5) | ((row == 0) & (i == 0))

    mid = jnp.dot(jax.nn.gelu(jnp.dot(h, w1_ref[...],
                                      preferred_element_type=f32)),
                  w2_ref[...], preferred_element_type=f32)
    yg = mid * probs

    rowf = row.astype(f32)
    c_col = jnp.where(mask, rowf, -1.0)
    rowi = jax.lax.broadcasted_iota(jnp.int32, (lb, lb), 0)
    colj = jax.lax.broadcasted_iota(jnp.int32, (lb, lb), 1)
    eye = (rowi == colj).astype(f32)
    c_row = jnp.dot(jnp.ones((1, lb), f32), eye * c_col,
                    preferred_element_type=f32)
    m_mat = jnp.where(colj <= rowi, jnp.broadcast_to(c_row, (lb, lb)), -1.0)
    b_col = jnp.max(m_mat, axis=1, keepdims=True)
    sel = (b_col == colj.astype(f32)).astype(f32)

    @pl.when(i == 0)
    def _():
        carry_ref[...] = jnp.zeros_like(carry_ref)

    carry_row = carry_ref[7:8, :]
    up = jnp.dot(sel, yg, preferred_element_type=f32)
    up = up + jnp.where(b_col < 0.0, carry_row, 0.0)
    carry_ref[...] = up[lb - 8:, :]

    fused = res + up
    out_ref[...] = jax.nn.gelu(jnp.dot(fused, wpost_ref[...],
                                       preferred_element_type=f32))


def kernel(hidden_states, x_pack_kwargs, W_pre, W_res, w_router, W1, W2,
           W_post):
    del x_pack_kwargs
    B, L, D = hidden_states.shape
    lb = 256
    x2d = hidden_states[0]
    wrt = w_router.reshape(D, 1)

    grid = (L // lb,)
    full = lambda a: pl.BlockSpec(a.shape, lambda i: (0,) * a.ndim)
    out = pl.pallas_call(
        functools.partial(_fused_block, lb=lb),
        grid=grid,
        in_specs=[
            pl.BlockSpec((lb, D), lambda i: (i, 0)),
            full(W_pre), full(W_res), full(wrt), full(W1), full(W2),
            full(W_post),
        ],
        out_specs=pl.BlockSpec((lb, D), lambda i: (i, 0)),
        out_shape=jax.ShapeDtypeStruct((L, D), jnp.float32),
        scratch_shapes=[pltpu.VMEM((8, D), jnp.float32)],
        compiler_params=pltpu.CompilerParams(
            dimension_semantics=("arbitrary",)),
    )(x2d, W_pre, W_res, wrt, W1, W2, W_post)
    return out[None]


# speedup vs baseline: 1.8788x; 1.0991x over previous
import functools

import jax
import jax.numpy as jnp
from jax.experimental import pallas as pl
from jax.experimental.pallas import tpu as pltpu


def _fused_block(x_ref, wpre_ref, wres_ref, wrt_ref, w1_ref, w2_ref,
                 wpost_ref, out_ref, carry_ref, *, lb):
    i = pl.program_id(0)
    f32 = jnp.float32

    x = x_ref[...]
    h = jax.nn.gelu(jnp.dot(x, wpre_ref[...], preferred_element_type=f32))
    h16 = h.astype(jnp.bfloat16)
    res = jnp.dot(h16, wres_ref[...], preferred_element_type=f32)
    logits = jnp.dot(h, wrt_ref[...], preferred_element_type=f32)
    probs = jax.nn.sigmoid(logits)

    row = jax.lax.broadcasted_iota(jnp.int32, (lb, 1), 0)
    mask = (probs >= 0.5) | ((row == 0) & (i == 0))

    t16 = jax.nn.gelu(jnp.dot(h16, w1_ref[...],
                              preferred_element_type=f32)).astype(jnp.bfloat16)
    mid = jnp.dot(t16, w2_ref[...], preferred_element_type=f32)
    yg = mid * probs

    rowf = row.astype(f32)
    c_col = jnp.where(mask, rowf, -1.0)
    rowi = jax.lax.broadcasted_iota(jnp.int32, (lb, lb), 0)
    colj = jax.lax.broadcasted_iota(jnp.int32, (lb, lb), 1)
    eye = (rowi == colj).astype(f32)
    c_row = jnp.dot(jnp.ones((1, lb), f32), eye * c_col,
                    preferred_element_type=f32)
    m_mat = jnp.where(colj <= rowi, jnp.broadcast_to(c_row, (lb, lb)), -1.0)
    b_col = jnp.max(m_mat, axis=1, keepdims=True)
    sel = (b_col == colj.astype(f32)).astype(f32)

    @pl.when(i == 0)
    def _():
        carry_ref[...] = jnp.zeros_like(carry_ref)

    carry_row = carry_ref[7:8, :]
    up = jnp.dot(sel, yg, preferred_element_type=f32)
    up = up + jnp.where(b_col < 0.0, carry_row, 0.0)
    carry_ref[...] = up[lb - 8:, :]

    fused16 = (res + up).astype(jnp.bfloat16)
    out_ref[...] = jax.nn.gelu(jnp.dot(fused16, wpost_ref[...],
                                       preferred_element_type=f32))


def kernel(hidden_states, x_pack_kwargs, W_pre, W_res, w_router, W1, W2,
           W_post):
    del x_pack_kwargs
    B, L, D = hidden_states.shape
    lb = 256
    x2d = hidden_states[0]
    wrt = w_router.reshape(D, 1)
    W_res = W_res.astype(jnp.bfloat16)
    W1 = W1.astype(jnp.bfloat16)
    W2 = W2.astype(jnp.bfloat16)
    W_post = W_post.astype(jnp.bfloat16)

    grid = (L // lb,)
    full = lambda a: pl.BlockSpec(a.shape, lambda i: (0,) * a.ndim)
    out = pl.pallas_call(
        functools.partial(_fused_block, lb=lb),
        grid=grid,
        in_specs=[
            pl.BlockSpec((lb, D), lambda i: (i, 0)),
            full(W_pre), full(W_res), full(wrt), full(W1), full(W2),
            full(W_post),
        ],
        out_specs=pl.BlockSpec((lb, D), lambda i: (i, 0)),
        out_shape=jax.ShapeDtypeStruct((L, D), jnp.float32),
        scratch_shapes=[pltpu.VMEM((8, D), jnp.float32)],
        compiler_params=pltpu.CompilerParams(
            dimension_semantics=("arbitrary",)),
    )(x2d, W_pre, W_res, wrt, W1, W2, W_post)
    return out[None]


# speedup vs baseline: 2.0773x; 1.1057x over previous
import functools

import jax
import jax.numpy as jnp
from jax.experimental import pallas as pl
from jax.experimental.pallas import tpu as pltpu


def _fused_block(x_ref, wpre_ref, wres_ref, wrt_ref, w1_ref, w2_ref,
                 wpost_ref, out_ref, carry_ref, wres16_ref, w116_ref,
                 w216_ref, wpost16_ref, *, lb):
    i = pl.program_id(0)
    f32 = jnp.float32
    bf16 = jnp.bfloat16

    @pl.when(i == 0)
    def _():
        carry_ref[...] = jnp.zeros_like(carry_ref)
        wres16_ref[...] = wres_ref[...].astype(bf16)
        w116_ref[...] = w1_ref[...].astype(bf16)
        w216_ref[...] = w2_ref[...].astype(bf16)
        wpost16_ref[...] = wpost_ref[...].astype(bf16)

    x = x_ref[...]
    h = jax.nn.gelu(jnp.dot(x, wpre_ref[...], preferred_element_type=f32))
    h16 = h.astype(bf16)
    res = jnp.dot(h16, wres16_ref[...], preferred_element_type=f32)
    logits = jnp.dot(h, wrt_ref[...], preferred_element_type=f32)
    probs = jax.nn.sigmoid(logits)

    row = jax.lax.broadcasted_iota(jnp.int32, (lb, 1), 0)
    mask = (probs >= 0.5) | ((row == 0) & (i == 0))

    t16 = jax.nn.gelu(jnp.dot(h16, w116_ref[...],
                              preferred_element_type=f32)).astype(bf16)
    mid = jnp.dot(t16, w216_ref[...], preferred_element_type=f32)
    yg = mid * probs

    rowf = row.astype(f32)
    c_col = jnp.where(mask, rowf, -1.0)
    rowi = jax.lax.broadcasted_iota(jnp.int32, (lb, lb), 0)
    colj = jax.lax.broadcasted_iota(jnp.int32, (lb, lb), 1)
    eye = (rowi == colj).astype(f32)
    c_row = jnp.dot(jnp.ones((1, lb), f32), eye * c_col,
                    preferred_element_type=f32)
    m_mat = jnp.where(colj <= rowi, jnp.broadcast_to(c_row, (lb, lb)), -1.0)
    b_col = jnp.max(m_mat, axis=1, keepdims=True)
    sel = (b_col == colj.astype(f32)).astype(f32)

    carry_row = carry_ref[7:8, :]
    up = jnp.dot(sel, yg, preferred_element_type=f32)
    up = up + jnp.where(b_col < 0.0, carry_row, 0.0)
    carry_ref[...] = up[lb - 8:, :]

    fused16 = (res + up).astype(bf16)
    out_ref[...] = jax.nn.gelu(jnp.dot(fused16, wpost16_ref[...],
                                       preferred_element_type=f32))


def kernel(hidden_states, x_pack_kwargs, W_pre, W_res, w_router, W1, W2,
           W_post):
    del x_pack_kwargs
    B, L, D = hidden_states.shape
    d_ff = W1.shape[1]
    lb = 256
    x2d = hidden_states[0]
    wrt = w_router.reshape(D, 1)

    grid = (L // lb,)
    full = lambda a: pl.BlockSpec(a.shape, lambda i: (0,) * a.ndim)
    out = pl.pallas_call(
        functools.partial(_fused_block, lb=lb),
        grid=grid,
        in_specs=[
            pl.BlockSpec((lb, D), lambda i: (i, 0)),
            full(W_pre), full(W_res), full(wrt), full(W1), full(W2),
            full(W_post),
        ],
        out_specs=pl.BlockSpec((lb, D), lambda i: (i, 0)),
        out_shape=jax.ShapeDtypeStruct((L, D), jnp.float32),
        scratch_shapes=[
            pltpu.VMEM((8, D), jnp.float32),
            pltpu.VMEM((D, D), jnp.bfloat16),
            pltpu.VMEM((D, d_ff), jnp.bfloat16),
            pltpu.VMEM((d_ff, D), jnp.bfloat16),
            pltpu.VMEM((D, D), jnp.bfloat16),
        ],
        compiler_params=pltpu.CompilerParams(
            dimension_semantics=("arbitrary",)),
    )(x2d, W_pre, W_res, wrt, W1, W2, W_post)
    return out[None]
